# Initial kernel scaffold; baseline (speedup 1.0000x reference)
import jax
import jax.numpy as jnp
from jax.experimental import pallas as pl


def kernel(inputs, emb_weight, pos):
    raise NotImplementedError("write your pallas kernel here")


# speedup vs baseline: 2.9817x; 2.9817x over previous
import functools

import jax
import jax.numpy as jnp
from jax import lax
from jax.experimental import pallas as pl
from jax.experimental.pallas import tpu as pltpu
from jax.experimental.pallas import tpu_sc as plsc

EMB = 128
SEQ = 200
BATCH = 4096
NC = 2
NS = 16
NW = NC * NS
ROWS = BATCH * SEQ
ROWS_PER_W = ROWS // NW
CHUNK = 128
CHUNKS = ROWS_PER_W // CHUNK
NBUF = 2
POS_EXT = SEQ + CHUNK
LANES = 16
EGRP = EMB // LANES


def _body(idx_hbm, table_hbm, pos_hbm, out_hbm,
          idx_v, pos_v, rows_v, gsem0, gsem1, wsem0, wsem1):
    gsems = (gsem0, gsem1)
    wsems = (wsem0, wsem1)
    wid = lax.axis_index("s") * NC + lax.axis_index("c")
    row0 = wid * ROWS_PER_W

    pltpu.sync_copy(pos_hbm, pos_v)
    pltpu.sync_copy(idx_hbm.at[pl.ds(wid * CHUNKS, CHUNKS)], idx_v)

    def start_gather(c, b):
        pltpu.async_copy(table_hbm.at[idx_v.at[c]], rows_v.at[b], gsems[b])

    start_gather(0, 0)
    start_gather(1, 1)

    def pair(g, carry):
        for b in range(NBUF):
            c = g * NBUF + b
            pltpu.make_async_copy(
                table_hbm.at[idx_v.at[c]], rows_v.at[b], gsems[b]).wait()
            s0 = lax.rem(c * CHUNK, SEQ)

            def add_row(j, acc):
                sr = s0 + j
                for e in range(EGRP):
                    p = pos_v[sr, pl.ds(e * LANES, LANES)]
                    plsc.addupdate(rows_v.at[b, j, pl.ds(e * LANES, LANES)], p)
                return acc

            lax.fori_loop(0, CHUNK, add_row, 0, unroll=2)

            dst = out_hbm.at[pl.ds(row0 + c * CHUNK, CHUNK)]
            pltpu.async_copy(rows_v.at[b], dst, wsems[b])
            pltpu.make_async_copy(rows_v.at[b], dst, wsems[b]).wait()

            @pl.when(c < CHUNKS - NBUF)
            def _():
                start_gather(c + NBUF, b)
        return carry

    lax.fori_loop(0, CHUNKS // NBUF, pair, 0)


_emb_kernel = functools.partial(
    pl.kernel,
    out_type=jax.ShapeDtypeStruct((ROWS, EMB), jnp.float32),
    mesh=plsc.VectorSubcoreMesh(core_axis_name="c", subcore_axis_name="s"),
    scratch_types=[
        pltpu.VMEM((CHUNKS, CHUNK), jnp.int32),
        pltpu.VMEM((POS_EXT, EMB), jnp.float32),
        pltpu.VMEM((NBUF, CHUNK, EMB), jnp.float32),
        pltpu.SemaphoreType.DMA,
        pltpu.SemaphoreType.DMA,
        pltpu.SemaphoreType.DMA,
        pltpu.SemaphoreType.DMA,
    ],
)(_body)


@jax.jit
def kernel(inputs, emb_weight, pos):
    idx = inputs.astype(jnp.int32).reshape(ROWS // CHUNK, CHUNK)
    pos_ext = jnp.concatenate([pos, pos[:CHUNK]], axis=0)
    out = _emb_kernel(idx, emb_weight, pos_ext)
    return out.reshape(BATCH, SEQ, EMB)


# speedup vs baseline: 7.3256x; 2.4569x over previous
import functools

import jax
import jax.numpy as jnp
from jax import lax
from jax.experimental import pallas as pl
from jax.experimental.pallas import tpu as pltpu
from jax.experimental.pallas import tpu_sc as plsc

EMB = 128
SEQ = 200
BATCH = 4096
NC = 2
NS = 16
NW = NC * NS
ROWS = BATCH * SEQ
ROWS_PER_W = ROWS // NW
CHUNK = 128
CHUNKS = ROWS_PER_W // CHUNK
NBUF = 2
POS_EXT = SEQ + CHUNK
LANES = 16
EGRP = EMB // LANES


def _body(idx_hbm, table_hbm, pos_hbm, out_hbm,
          idx_v, pos_v, rows_v, gsem0, gsem1, wsem0, wsem1):
    gsems = (gsem0, gsem1)
    wsems = (wsem0, wsem1)
    wid = lax.axis_index("s") * NC + lax.axis_index("c")
    row0 = wid * ROWS_PER_W

    pltpu.sync_copy(pos_hbm, pos_v)
    pltpu.sync_copy(idx_hbm.at[pl.ds(wid * CHUNKS, CHUNKS)], idx_v)

    def start_gather(c, b):
        pltpu.async_copy(table_hbm.at[idx_v.at[c]], rows_v.at[b], gsems[b])

    start_gather(0, 0)
    start_gather(1, 1)

    def pair(g, carry):
        for b in range(NBUF):
            c = g * NBUF + b
            pltpu.make_async_copy(
                table_hbm.at[idx_v.at[c]], rows_v.at[b], gsems[b]).wait()
            s0 = lax.rem(c * CHUNK, SEQ)

            def add_row(j, acc):
                base = (s0 + j) * EMB
                ps = [pos_v[pl.ds(base + e * LANES, LANES)]
                      for e in range(EGRP)]
                for e in range(EGRP):
                    plsc.addupdate(
                        rows_v.at[b, j, pl.ds(e * LANES, LANES)], ps[e])
                return acc

            lax.fori_loop(0, CHUNK, add_row, 0, unroll=2)

            dst = out_hbm.at[pl.ds(row0 + c * CHUNK, CHUNK)]
            pltpu.async_copy(rows_v.at[b], dst, wsems[b])
            pltpu.make_async_copy(rows_v.at[b], dst, wsems[b]).wait()

            @pl.when(c < CHUNKS - NBUF)
            def _():
                start_gather(c + NBUF, b)
        return carry

    lax.fori_loop(0, CHUNKS // NBUF, pair, 0)


_emb_kernel = functools.partial(
    pl.kernel,
    out_type=jax.ShapeDtypeStruct((ROWS, EMB), jnp.float32),
    mesh=plsc.VectorSubcoreMesh(core_axis_name="c", subcore_axis_name="s"),
    scratch_types=[
        pltpu.VMEM((CHUNKS, CHUNK), jnp.int32),
        pltpu.VMEM((POS_EXT * EMB,), jnp.float32),
        pltpu.VMEM((NBUF, CHUNK, EMB), jnp.float32),
        pltpu.SemaphoreType.DMA,
        pltpu.SemaphoreType.DMA,
        pltpu.SemaphoreType.DMA,
        pltpu.SemaphoreType.DMA,
    ],
)(_body)


@jax.jit
def kernel(inputs, emb_weight, pos):
    idx = inputs.astype(jnp.int32).reshape(ROWS // CHUNK, CHUNK)
    pos_ext = jnp.concatenate([pos, pos[:CHUNK]], axis=0).reshape(-1)
    out = _emb_kernel(idx, emb_weight, pos_ext)
    return out.reshape(BATCH, SEQ, EMB)
